# baseline (device time: 149951 ns/iter reference)
import jax
import jax.numpy as jnp
from jax import lax
from jax.experimental import pallas as pl
from jax.experimental.pallas import tpu as pltpu

N_DEV = 16
N_TOK = 2048
D_IN = 512
D_OUT = 1024
E_LOCAL = 4
CHUNK = N_TOK // N_DEV


def kernel(x, router_W, route_idx, expert_W):
    def body(x_ref, rw_ref, idx_ref, ew_ref, out_ref,
             partial_ref, send_ref, recv_ref, send_sem, recv_sems):
        my = lax.axis_index("i")
        left = lax.rem(my - 1 + N_DEV, N_DEV)
        right = lax.rem(my + 1, N_DEV)

        barrier = pltpu.get_barrier_semaphore()
        pl.semaphore_signal(barrier, inc=1, device_id=(left,),
                            device_id_type=pl.DeviceIdType.MESH)
        pl.semaphore_signal(barrier, inc=1, device_id=(right,),
                            device_id_type=pl.DeviceIdType.MESH)
        pl.semaphore_wait(barrier, 2)

        xs = x_ref[:, :]
        scores = jnp.dot(xs, rw_ref[:, :],
                         preferred_element_type=jnp.float32)
        r0 = idx_ref[:, 0:1]
        r1 = idx_ref[:, 1:2]
        eids = lax.broadcasted_iota(jnp.int32, (1, 64), 1)
        s0 = jnp.sum(scores * (r0 == eids).astype(jnp.float32),
                     axis=1, keepdims=True)
        s1 = jnp.sum(scores * (r1 == eids).astype(jnp.float32),
                     axis=1, keepdims=True)
        m = jnp.maximum(s0, s1)
        p0 = jnp.exp(s0 - m)
        p1 = jnp.exp(s1 - m)
        g0 = p0 / (p0 + p1)
        g1 = p1 / (p0 + p1)

        acc = jnp.zeros((N_TOK, D_OUT), jnp.float32)
        for j in range(E_LOCAL):
            e_glob = my * E_LOCAL + j
            gate = (jnp.where(r0 == e_glob, g0, 0.0)
                    + jnp.where(r1 == e_glob, g1, 0.0))
            acc = acc + jnp.dot(xs * gate, ew_ref[j],
                                preferred_element_type=jnp.float32)
        partial_ref[:, :] = acc

        def chunk_rows(i):
            c = lax.rem(i + 8 * N_DEV, N_DEV)
            return pl.ds(c * CHUNK, CHUNK)

        send_ref[:, :] = partial_ref[chunk_rows(my - 1), :]
        for s in range(N_DEV - 1):
            rdma = pltpu.make_async_remote_copy(
                src_ref=send_ref,
                dst_ref=recv_ref.at[s],
                send_sem=send_sem,
                recv_sem=recv_sems.at[s],
                device_id=(right,),
                device_id_type=pl.DeviceIdType.MESH,
            )
            rdma.start()
            rdma.wait()
            if s < N_DEV - 2:
                send_ref[:, :] = (recv_ref[s]
                                  + partial_ref[chunk_rows(my - 2 - s), :])
            else:
                out_ref[:, :] = recv_ref[s] + partial_ref[chunk_rows(my), :]

    return pl.pallas_call(
        body,
        out_shape=jax.ShapeDtypeStruct((CHUNK, D_OUT), jnp.float32),
        in_specs=[
            pl.BlockSpec(memory_space=pltpu.VMEM),
            pl.BlockSpec(memory_space=pltpu.VMEM),
            pl.BlockSpec(memory_space=pltpu.VMEM),
            pl.BlockSpec(memory_space=pltpu.VMEM),
        ],
        out_specs=pl.BlockSpec(memory_space=pltpu.VMEM),
        scratch_shapes=[
            pltpu.VMEM((N_TOK, D_OUT), jnp.float32),
            pltpu.VMEM((CHUNK, D_OUT), jnp.float32),
            pltpu.VMEM((N_DEV - 1, CHUNK, D_OUT), jnp.float32),
            pltpu.SemaphoreType.DMA,
            pltpu.SemaphoreType.DMA((N_DEV - 1,)),
        ],
        compiler_params=pltpu.CompilerParams(collective_id=0),
    )(x, router_W, route_idx, expert_W)


# device time: 88332 ns/iter; 1.6976x vs baseline; 1.6976x over previous
import jax
import jax.numpy as jnp
from jax import lax
from jax.experimental import pallas as pl
from jax.experimental.pallas import tpu as pltpu

N_DEV = 16
N_TOK = 2048
D_IN = 512
D_OUT = 1024
E_LOCAL = 4
CHUNK = N_TOK // N_DEV

WIRE_DTYPE = jnp.bfloat16


def kernel(x, router_W, route_idx, expert_W):
    def body(x_ref, rw_ref, idx_ref, ew_ref, out_ref,
             gates_ref, pchunk_ref, send_ref, recv_ref, send_sems, recv_sems):
        my = lax.axis_index("i")
        left = lax.rem(my - 1 + N_DEV, N_DEV)
        right = lax.rem(my + 1, N_DEV)

        barrier = pltpu.get_barrier_semaphore()
        pl.semaphore_signal(barrier, inc=1, device_id=(left,),
                            device_id_type=pl.DeviceIdType.MESH)
        pl.semaphore_signal(barrier, inc=1, device_id=(right,),
                            device_id_type=pl.DeviceIdType.MESH)
        pl.semaphore_wait(barrier, 2)

        scores = jnp.dot(x_ref[:, :], rw_ref[:, :],
                         preferred_element_type=jnp.float32)
        r0 = idx_ref[:, 0:1]
        r1 = idx_ref[:, 1:2]
        eids = lax.broadcasted_iota(jnp.int32, (1, 64), 1)
        s0 = jnp.sum(scores * (r0 == eids).astype(jnp.float32),
                     axis=1, keepdims=True)
        s1 = jnp.sum(scores * (r1 == eids).astype(jnp.float32),
                     axis=1, keepdims=True)
        m = jnp.maximum(s0, s1)
        p0 = jnp.exp(s0 - m)
        p1 = jnp.exp(s1 - m)
        g0 = p0 / (p0 + p1)
        g1 = p1 / (p0 + p1)

        for j in range(E_LOCAL):
            e_glob = my * E_LOCAL + j
            gates_ref[:, j:j + 1] = (jnp.where(r0 == e_glob, g0, 0.0)
                                     + jnp.where(r1 == e_glob, g1, 0.0))

        def compute_chunk(i):
            start = lax.rem(i + 8 * N_DEV, N_DEV) * CHUNK
            xc = x_ref[pl.ds(start, CHUNK), :]
            acc = jnp.zeros((CHUNK, D_OUT), jnp.float32)
            for j in range(E_LOCAL):
                gate = gates_ref[pl.ds(start, CHUNK), j:j + 1]
                acc = acc + jnp.dot(xc * gate, ew_ref[j],
                                    preferred_element_type=jnp.float32)
            return acc

        def make_rdma(s):
            return pltpu.make_async_remote_copy(
                src_ref=send_ref.at[s % 2],
                dst_ref=recv_ref.at[s],
                send_sem=send_sems.at[s % 2],
                recv_sem=recv_sems.at[s],
                device_id=(right,),
                device_id_type=pl.DeviceIdType.MESH,
            )

        send_ref[0, :, :] = compute_chunk(my - 1).astype(WIRE_DTYPE)
        rdmas = [make_rdma(s) for s in range(N_DEV - 1)]
        rdmas[0].start()
        for s in range(N_DEV - 1):
            pchunk_ref[:, :] = compute_chunk(my - 2 - s)
            rdmas[s].wait_recv()
            acc = recv_ref[s].astype(jnp.float32) + pchunk_ref[:, :]
            if s < N_DEV - 2:
                nslot = (s + 1) % 2
                if s >= 1:
                    rdmas[s - 1].wait_send()
                send_ref[nslot, :, :] = acc.astype(WIRE_DTYPE)
                rdmas[s + 1].start()
            else:
                out_ref[:, :] = acc
        rdmas[N_DEV - 3].wait_send()
        rdmas[N_DEV - 2].wait_send()

    return pl.pallas_call(
        body,
        out_shape=jax.ShapeDtypeStruct((CHUNK, D_OUT), jnp.float32),
        in_specs=[
            pl.BlockSpec(memory_space=pltpu.VMEM),
            pl.BlockSpec(memory_space=pltpu.VMEM),
            pl.BlockSpec(memory_space=pltpu.VMEM),
            pl.BlockSpec(memory_space=pltpu.VMEM),
        ],
        out_specs=pl.BlockSpec(memory_space=pltpu.VMEM),
        scratch_shapes=[
            pltpu.VMEM((N_TOK, E_LOCAL), jnp.float32),
            pltpu.VMEM((CHUNK, D_OUT), jnp.float32),
            pltpu.VMEM((2, CHUNK, D_OUT), WIRE_DTYPE),
            pltpu.VMEM((N_DEV - 1, CHUNK, D_OUT), WIRE_DTYPE),
            pltpu.SemaphoreType.DMA((2,)),
            pltpu.SemaphoreType.DMA((N_DEV - 1,)),
        ],
        compiler_params=pltpu.CompilerParams(collective_id=0),
    )(x, router_W, route_idx, expert_W)


# device time: 87407 ns/iter; 1.7155x vs baseline; 1.0106x over previous
import jax
import jax.numpy as jnp
from jax import lax
from jax.experimental import pallas as pl
from jax.experimental.pallas import tpu as pltpu

N_DEV = 16
N_TOK = 2048
D_IN = 512
D_OUT = 1024
H_HALF = D_OUT // 2
E_LOCAL = 4
CHUNK = N_TOK // N_DEV

WIRE_DTYPE = jnp.bfloat16


def kernel(x, router_W, route_idx, expert_W):
    def body(x_ref, rw_ref, idx_ref, ew_ref, out_ref,
             gates_ref, pr_ref, pl_ref,
             sendr_ref, sendl_ref, recvr_ref, recvl_ref,
             sendr_sems, sendl_sems, recvr_sems, recvl_sems):
        my = lax.axis_index("i")
        left = lax.rem(my - 1 + N_DEV, N_DEV)
        right = lax.rem(my + 1, N_DEV)

        barrier = pltpu.get_barrier_semaphore()
        pl.semaphore_signal(barrier, inc=1, device_id=(left,),
                            device_id_type=pl.DeviceIdType.MESH)
        pl.semaphore_signal(barrier, inc=1, device_id=(right,),
                            device_id_type=pl.DeviceIdType.MESH)
        pl.semaphore_wait(barrier, 2)

        scores = jnp.dot(x_ref[:, :], rw_ref[:, :],
                         preferred_element_type=jnp.float32)
        r0 = idx_ref[:, 0:1]
        r1 = idx_ref[:, 1:2]
        eids = lax.broadcasted_iota(jnp.int32, (1, 64), 1)
        s0 = jnp.sum(scores * (r0 == eids).astype(jnp.float32),
                     axis=1, keepdims=True)
        s1 = jnp.sum(scores * (r1 == eids).astype(jnp.float32),
                     axis=1, keepdims=True)
        m = jnp.maximum(s0, s1)
        p0 = jnp.exp(s0 - m)
        p1 = jnp.exp(s1 - m)
        g0 = p0 / (p0 + p1)
        g1 = p1 / (p0 + p1)

        for j in range(E_LOCAL):
            e_glob = my * E_LOCAL + j
            gates_ref[:, j:j + 1] = (jnp.where(r0 == e_glob, g0, 0.0)
                                     + jnp.where(r1 == e_glob, g1, 0.0))

        def half_chunk(i, col0):
            start = lax.rem(i + 8 * N_DEV, N_DEV) * CHUNK
            xc = x_ref[pl.ds(start, CHUNK), :]
            acc = jnp.zeros((CHUNK, H_HALF), jnp.float32)
            for j in range(E_LOCAL):
                gate = gates_ref[pl.ds(start, CHUNK), j:j + 1]
                acc = acc + jnp.dot(xc * gate,
                                    ew_ref[j, :, col0:col0 + H_HALF],
                                    preferred_element_type=jnp.float32)
            return acc

        def make_rdma(s, send_ref, recv_ref, send_sems, recv_sems, tgt):
            return pltpu.make_async_remote_copy(
                src_ref=send_ref.at[s % 2],
                dst_ref=recv_ref.at[s],
                send_sem=send_sems.at[s % 2],
                recv_sem=recv_sems.at[s],
                device_id=(tgt,),
                device_id_type=pl.DeviceIdType.MESH,
            )

        sendr_ref[0, :, :] = half_chunk(my - 1, 0).astype(WIRE_DTYPE)
        sendl_ref[0, :, :] = half_chunk(my + 1, H_HALF).astype(WIRE_DTYPE)
        rdmas_r = [make_rdma(s, sendr_ref, recvr_ref, sendr_sems,
                             recvr_sems, right) for s in range(N_DEV - 1)]
        rdmas_l = [make_rdma(s, sendl_ref, recvl_ref, sendl_sems,
                             recvl_sems, left) for s in range(N_DEV - 1)]
        rdmas_r[0].start()
        rdmas_l[0].start()
        for s in range(N_DEV - 1):
            pr_ref[:, :] = half_chunk(my - 2 - s, 0)
            pl_ref[:, :] = half_chunk(my + 2 + s, H_HALF)
            rdmas_r[s].wait_recv()
            rdmas_l[s].wait_recv()
            acc_r = recvr_ref[s].astype(jnp.float32) + pr_ref[:, :]
            acc_l = recvl_ref[s].astype(jnp.float32) + pl_ref[:, :]
            if s < N_DEV - 2:
                nslot = (s + 1) % 2
                if s >= 1:
                    rdmas_r[s - 1].wait_send()
                    rdmas_l[s - 1].wait_send()
                sendr_ref[nslot, :, :] = acc_r.astype(WIRE_DTYPE)
                sendl_ref[nslot, :, :] = acc_l.astype(WIRE_DTYPE)
                rdmas_r[s + 1].start()
                rdmas_l[s + 1].start()
            else:
                out_ref[:, 0:H_HALF] = acc_r
                out_ref[:, H_HALF:D_OUT] = acc_l
        for rd in (rdmas_r, rdmas_l):
            rd[N_DEV - 3].wait_send()
            rd[N_DEV - 2].wait_send()

    return pl.pallas_call(
        body,
        out_shape=jax.ShapeDtypeStruct((CHUNK, D_OUT), jnp.float32),
        in_specs=[
            pl.BlockSpec(memory_space=pltpu.VMEM),
            pl.BlockSpec(memory_space=pltpu.VMEM),
            pl.BlockSpec(memory_space=pltpu.VMEM),
            pl.BlockSpec(memory_space=pltpu.VMEM),
        ],
        out_specs=pl.BlockSpec(memory_space=pltpu.VMEM),
        scratch_shapes=[
            pltpu.VMEM((N_TOK, E_LOCAL), jnp.float32),
            pltpu.VMEM((CHUNK, H_HALF), jnp.float32),
            pltpu.VMEM((CHUNK, H_HALF), jnp.float32),
            pltpu.VMEM((2, CHUNK, H_HALF), WIRE_DTYPE),
            pltpu.VMEM((2, CHUNK, H_HALF), WIRE_DTYPE),
            pltpu.VMEM((N_DEV - 1, CHUNK, H_HALF), WIRE_DTYPE),
            pltpu.VMEM((N_DEV - 1, CHUNK, H_HALF), WIRE_DTYPE),
            pltpu.SemaphoreType.DMA((2,)),
            pltpu.SemaphoreType.DMA((2,)),
            pltpu.SemaphoreType.DMA((N_DEV - 1,)),
            pltpu.SemaphoreType.DMA((N_DEV - 1,)),
        ],
        compiler_params=pltpu.CompilerParams(collective_id=0),
    )(x, router_W, route_idx, expert_W)


# device time: 54913 ns/iter; 2.7307x vs baseline; 1.5917x over previous
import jax
import jax.numpy as jnp
from jax import lax
from jax.experimental import pallas as pl
from jax.experimental.pallas import tpu as pltpu

N_DEV = 16
N_TOK = 2048
D_IN = 512
D_OUT = 1024
H_HALF = D_OUT // 2
E_LOCAL = 4
CHUNK = N_TOK // N_DEV
GROUP = 4 * CHUNK

WIRE_DTYPE = jnp.bfloat16


def kernel(x, router_W, route_idx, expert_W):
    ew2 = expert_W.reshape(E_LOCAL * D_IN, D_OUT)

    def body(x_ref, rw_ref, idx_ref, ew_ref, out_ref,
             gates_ref, par_ref, pal_ref,
             stap_ref, stam_ref, stbp_ref, stbm_ref,
             rap_ref, ram_ref, rbp_ref, rbm_ref,
             sap_sem, sam_sem, sbp_sem, sbm_sem,
             rap_sems, ram_sems, rbp_sems, rbm_sems):
        my = lax.axis_index("i")
        k = lax.rem(my, 4)
        z = lax.div(my, 4)

        def m4(v):
            return lax.rem(v + 8, 4)

        kp1 = z * 4 + m4(k + 1)
        km1 = z * 4 + m4(k - 1)
        zp1 = m4(z + 1) * 4 + k
        zm1 = m4(z - 1) * 4 + k

        barrier = pltpu.get_barrier_semaphore()
        for nbr in (kp1, km1, zp1, zm1):
            pl.semaphore_signal(barrier, inc=1, device_id=(nbr,),
                                device_id_type=pl.DeviceIdType.MESH)
        pl.semaphore_wait(barrier, 4)

        scores = jnp.dot(x_ref[:, :], rw_ref[:, :],
                         preferred_element_type=jnp.float32)
        r0 = idx_ref[:, 0:1]
        r1 = idx_ref[:, 1:2]
        eids = lax.broadcasted_iota(jnp.int32, (1, 64), 1)
        s0 = jnp.sum(scores * (r0 == eids).astype(jnp.float32),
                     axis=1, keepdims=True)
        s1 = jnp.sum(scores * (r1 == eids).astype(jnp.float32),
                     axis=1, keepdims=True)
        m = jnp.maximum(s0, s1)
        p0 = jnp.exp(s0 - m)
        p1 = jnp.exp(s1 - m)
        g0 = p0 / (p0 + p1)
        g1 = p1 / (p0 + p1)

        for j in range(E_LOCAL):
            e_glob = my * E_LOCAL + j
            gates_ref[:, j:j + 1] = (jnp.where(r0 == e_glob, g0, 0.0)
                                     + jnp.where(r1 == e_glob, g1, 0.0))

        def group_half(g, col0):
            parts = []
            for zp in range(4):
                row0 = (4 * zp + g) * CHUNK
                xc = x_ref[pl.ds(row0, CHUNK), :]
                parts.append(jnp.concatenate(
                    [xc * gates_ref[pl.ds(row0, CHUNK), j:j + 1]
                     for j in range(E_LOCAL)], axis=1))
            xcat = jnp.concatenate(parts, axis=0)
            return jnp.dot(xcat, ew_ref[:, col0:col0 + H_HALF],
                           preferred_element_type=jnp.float32)

        def mk(src_ref, dst_ref, ssem, rsem, tgt):
            return pltpu.make_async_remote_copy(
                src_ref=src_ref, dst_ref=dst_ref, send_sem=ssem,
                recv_sem=rsem, device_id=(tgt,),
                device_id_type=pl.DeviceIdType.MESH)

        a_p = [mk(stap_ref, rap_ref.at[s], sap_sem, rap_sems.at[s], kp1)
               for s in range(3)]
        a_m = [mk(stam_ref, ram_ref.at[s], sam_sem, ram_sems.at[s], km1)
               for s in range(3)]
        b_p = [mk(stbp_ref, rbp_ref.at[s], sbp_sem, rbp_sems.at[s], zp1)
               for s in range(3)]
        b_m = [mk(stbm_ref, rbm_ref.at[s], sbm_sem, rbm_sems.at[s], zm1)
               for s in range(3)]

        f32 = jnp.float32

        stap_ref[:, :] = group_half(m4(k - 1), 0).astype(WIRE_DTYPE)
        a_p[0].start()
        stam_ref[:, :] = group_half(m4(k + 1), H_HALF).astype(WIRE_DTYPE)
        a_m[0].start()

        pr1 = group_half(m4(k + 2), 0)
        pl1 = group_half(m4(k + 2), H_HALF)
        a_p[0].wait_recv()
        a_p[0].wait_send()
        stap_ref[:, :] = (rap_ref[0].astype(f32) + pr1).astype(WIRE_DTYPE)
        a_p[1].start()
        a_m[0].wait_recv()
        a_m[0].wait_send()
        stam_ref[:, :] = (ram_ref[0].astype(f32) + pl1).astype(WIRE_DTYPE)
        a_m[1].start()

        pr2 = group_half(m4(k + 1), 0)
        pl2 = group_half(m4(k - 1), H_HALF)
        a_p[1].wait_recv()
        a_p[1].wait_send()
        stap_ref[:, :] = (rap_ref[1].astype(f32) + pr2).astype(WIRE_DTYPE)
        a_p[2].start()
        a_m[1].wait_recv()
        a_m[1].wait_send()
        stam_ref[:, :] = (ram_ref[1].astype(f32) + pl2).astype(WIRE_DTYPE)
        a_m[2].start()

        prf = group_half(k, 0)
        plf = group_half(k, H_HALF)
        a_p[2].wait_recv()
        par_ref[:, :] = rap_ref[2].astype(f32) + prf
        a_m[2].wait_recv()
        pal_ref[:, :] = ram_ref[2].astype(f32) + plf

        stbp_ref[:, :] = par_ref[pl.ds(m4(z - 1) * CHUNK, CHUNK), :] \
            .astype(WIRE_DTYPE)
        b_p[0].start()
        stbm_ref[:, :] = pal_ref[pl.ds(m4(z + 1) * CHUNK, CHUNK), :] \
            .astype(WIRE_DTYPE)
        b_m[0].start()
        for s in range(3):
            b_p[s].wait_recv()
            acc_p = (rbp_ref[s].astype(f32)
                     + par_ref[pl.ds(m4(z - 2 - s) * CHUNK, CHUNK), :])
            if s < 2:
                b_p[s].wait_send()
                stbp_ref[:, :] = acc_p.astype(WIRE_DTYPE)
                b_p[s + 1].start()
            else:
                out_ref[:, 0:H_HALF] = acc_p
            b_m[s].wait_recv()
            acc_m = (rbm_ref[s].astype(f32)
                     + pal_ref[pl.ds(m4(z + 2 + s) * CHUNK, CHUNK), :])
            if s < 2:
                b_m[s].wait_send()
                stbm_ref[:, :] = acc_m.astype(WIRE_DTYPE)
                b_m[s + 1].start()
            else:
                out_ref[:, H_HALF:D_OUT] = acc_m

        a_p[2].wait_send()
        a_m[2].wait_send()
        b_p[2].wait_send()
        b_m[2].wait_send()

    return pl.pallas_call(
        body,
        out_shape=jax.ShapeDtypeStruct((CHUNK, D_OUT), jnp.float32),
        in_specs=[
            pl.BlockSpec(memory_space=pltpu.VMEM),
            pl.BlockSpec(memory_space=pltpu.VMEM),
            pl.BlockSpec(memory_space=pltpu.VMEM),
            pl.BlockSpec(memory_space=pltpu.VMEM),
        ],
        out_specs=pl.BlockSpec(memory_space=pltpu.VMEM),
        scratch_shapes=[
            pltpu.VMEM((N_TOK, E_LOCAL), jnp.float32),
            pltpu.VMEM((GROUP, H_HALF), jnp.float32),
            pltpu.VMEM((GROUP, H_HALF), jnp.float32),
            pltpu.VMEM((GROUP, H_HALF), WIRE_DTYPE),
            pltpu.VMEM((GROUP, H_HALF), WIRE_DTYPE),
            pltpu.VMEM((CHUNK, H_HALF), WIRE_DTYPE),
            pltpu.VMEM((CHUNK, H_HALF), WIRE_DTYPE),
            pltpu.VMEM((3, GROUP, H_HALF), WIRE_DTYPE),
            pltpu.VMEM((3, GROUP, H_HALF), WIRE_DTYPE),
            pltpu.VMEM((3, CHUNK, H_HALF), WIRE_DTYPE),
            pltpu.VMEM((3, CHUNK, H_HALF), WIRE_DTYPE),
            pltpu.SemaphoreType.DMA,
            pltpu.SemaphoreType.DMA,
            pltpu.SemaphoreType.DMA,
            pltpu.SemaphoreType.DMA,
            pltpu.SemaphoreType.DMA((3,)),
            pltpu.SemaphoreType.DMA((3,)),
            pltpu.SemaphoreType.DMA((3,)),
            pltpu.SemaphoreType.DMA((3,)),
        ],
        compiler_params=pltpu.CompilerParams(collective_id=0),
    )(x, router_W, route_idx, ew2)
